# baseline (speedup 1.0000x reference)
import functools

import jax
import jax.numpy as jnp
from jax import lax
from jax.experimental import pallas as pl
from jax.experimental.pallas import tpu as pltpu
from jax.experimental.pallas import tpu_sc as plsc

N_NODES = 10000
N_PAD = 10240
NC, NS = 2, 16
NW = NC * NS
CHUNK = 112
RB = 512
MAX_DEG = 5


NRING = 3


def _make_seg_sum(D, cpt0, cpt1):
  groups0, groups1 = cpt0 // NRING, cpt1 // NRING
  rows_per_tile = N_PAD // NS
  mesh = plsc.VectorSubcoreMesh(
      core_axis_name="c", subcore_axis_name="s", num_cores=NC, num_subcores=NS)

  @functools.partial(
      pl.kernel,
      out_type=(jax.ShapeDtypeStruct((N_PAD, D), jnp.float32),
                jax.ShapeDtypeStruct((N_PAD, D), jnp.float32)),
      mesh=mesh,
      compiler_params=pltpu.CompilerParams(use_tc_tiling_on_sc=False),
      scratch_types=[
          pltpu.VMEM_SHARED((N_PAD, D), jnp.float32),
          pltpu.VMEM((2, NRING, CHUNK), jnp.int32),
          pltpu.VMEM((2, NRING, CHUNK), jnp.int32),
          pltpu.VMEM((NRING, CHUNK, D), jnp.float32),
          pltpu.SemaphoreType.DMA,
          pltpu.SemaphoreType.DMA,
          pltpu.SemaphoreType.DMA,
      ],
  )
  def seg(x_hbm, src_hbm, dst_hbm, zero_hbm, h0_hbm, h1_hbm,
          hsh, sidx, didx, ring, gsem, ssem, isem):
    c = lax.axis_index("c")
    s = lax.axis_index("s")
    r0 = s * rows_per_tile
    g0row = jnp.where(c == 0, s * cpt0, NS * cpt0 + s * cpt1)
    groups = jnp.where(c == 0, groups0, groups1)

    pltpu.sync_copy(zero_hbm.at[pl.ds(r0, rows_per_tile)],
                    hsh.at[pl.ds(r0, rows_per_tile)])
    plsc.subcore_barrier()

    def idx_fetch(g, buf):
      row = g0row + g * NRING
      pltpu.async_copy(src_hbm.at[pl.ds(row, NRING)], sidx.at[buf], isem)
      pltpu.async_copy(dst_hbm.at[pl.ds(row, NRING)], didx.at[buf], isem)

    def idx_wait(buf):
      pltpu.make_async_copy(src_hbm.at[pl.ds(g0row, NRING)],
                            sidx.at[buf], isem).wait()
      pltpu.make_async_copy(dst_hbm.at[pl.ds(g0row, NRING)],
                            didx.at[buf], isem).wait()

    def gather(buf, j):
      pltpu.async_copy(x_hbm.at[sidx.at[buf, j]], ring.at[j], gsem)

    def gather_wait(j):
      pltpu.make_async_copy(x_hbm.at[sidx.at[0, 0]], ring.at[j], gsem).wait()

    def scat(buf, j):
      pltpu.async_copy(ring.at[j], hsh.at[didx.at[buf, j]], ssem, add=True)

    def scat_wait(j):
      pltpu.make_async_copy(ring.at[j], hsh.at[didx.at[0, 0]], ssem).wait()

    idx_fetch(0, 0)
    idx_wait(0)
    for j in range(NRING):
      gather(0, j)
    idx_fetch(1, 1)

    def body(g, carry):
      cb = lax.rem(g, 2)
      nb = 1 - cb
      for j in range(NRING):
        gather_wait(j)
        scat(cb, j)

      @pl.when(g + 1 < groups)
      def _():
        idx_wait(nb)
        for j in range(NRING):
          scat_wait(j)
          gather(nb, j)

        @pl.when(g + 2 < groups)
        def _():
          idx_fetch(g + 2, cb)

      return carry

    lax.fori_loop(0, groups, body, 0)
    for j in range(NRING):
      scat_wait(j)
    plsc.subcore_barrier()

    @pl.when(c == 0)
    def _():
      pltpu.sync_copy(hsh.at[pl.ds(r0, rows_per_tile)],
                      h0_hbm.at[pl.ds(r0, rows_per_tile)])

    @pl.when(c == 1)
    def _():
      pltpu.sync_copy(hsh.at[pl.ds(r0, rows_per_tile)],
                      h1_hbm.at[pl.ds(r0, rows_per_tile)])

  return seg


def _bucketed(h, x, degb, wl_ref, bl_ref, wr_ref):
  acc = jnp.zeros((h.shape[0], wl_ref.shape[2]), jnp.float32)
  for d in range(MAX_DEG + 1):
    r = (jnp.dot(h, wl_ref[d], preferred_element_type=jnp.float32)
         + jnp.dot(x, wr_ref[d], preferred_element_type=jnp.float32)
         + bl_ref[d][None, :])
    acc = jnp.where(degb == float(d), r, acc)
  return acc


def _layer1_body(h0, h1, xp, wl, bl, wr, out, degb_out):
  h = h0[...] + h1[...]
  degf = jnp.minimum(h[:, 29:30], float(MAX_DEG))
  degb = jnp.broadcast_to(degf, (h.shape[0], 128))
  out[...] = _bucketed(h, xp[...], degb, wl, bl, wr)
  degb_out[...] = degb


def _layer_mid_body(h0, h1, x, degb, wl, bl, wr, out):
  h = h0[...] + h1[...]
  out[...] = _bucketed(h, x[...], degb[...], wl, bl, wr)


def _layer4_body(h0, h1, x, degb, res2, wl, bl, wr, w1, b1, w2p, b2p, out):
  h = h0[...] + h1[...]
  res = _bucketed(h, x[...], degb[...], wl, bl, wr) + x[...] + res2[...]
  t = jnp.maximum(
      jnp.dot(res, w1[...], preferred_element_type=jnp.float32) + b1[...], 0.0)
  out[...] = (jnp.dot(t, w2p[...], preferred_element_type=jnp.float32)
              + b2p[...])


def _row_spec(dcols):
  return pl.BlockSpec((RB, dcols), lambda g: (g, 0))


def _full_spec(shape):
  nd = len(shape)
  return pl.BlockSpec(shape, lambda g: (0,) * nd)


def _tc_layer1(h0, h1, xp, wl, bl, wr):
  grid = (N_PAD // RB,)
  return pl.pallas_call(
      _layer1_body,
      grid=grid,
      in_specs=[_row_spec(32), _row_spec(32), _row_spec(32),
                _full_spec(wl.shape), _full_spec(bl.shape),
                _full_spec(wr.shape)],
      out_specs=[_row_spec(128), _row_spec(128)],
      out_shape=[jax.ShapeDtypeStruct((N_PAD, 128), jnp.float32)] * 2,
  )(h0, h1, xp, wl, bl, wr)


def _tc_layer_mid(h0, h1, x, degb, wl, bl, wr):
  grid = (N_PAD // RB,)
  return pl.pallas_call(
      _layer_mid_body,
      grid=grid,
      in_specs=[_row_spec(128)] * 4 + [_full_spec(wl.shape),
                                       _full_spec(bl.shape),
                                       _full_spec(wr.shape)],
      out_specs=_row_spec(128),
      out_shape=jax.ShapeDtypeStruct((N_PAD, 128), jnp.float32),
  )(h0, h1, x, degb, wl, bl, wr)


def _tc_layer4(h0, h1, x, degb, res2, wl, bl, wr, w1, b1, w2p, b2p):
  grid = (N_PAD // RB,)
  return pl.pallas_call(
      _layer4_body,
      grid=grid,
      in_specs=[_row_spec(128)] * 5 + [
          _full_spec(wl.shape), _full_spec(bl.shape), _full_spec(wr.shape),
          _full_spec(w1.shape), _full_spec(b1.shape), _full_spec(w2p.shape),
          _full_spec(b2p.shape)],
      out_specs=_row_spec(128),
      out_shape=jax.ShapeDtypeStruct((N_PAD, 128), jnp.float32),
  )(h0, h1, x, degb, res2, wl, bl, wr, w1, b1, w2p, b2p)


def kernel(x, edge_index, params):
  n, in_feat = x.shape
  e = edge_index.shape[1]
  t_chunks = -(-e // CHUNK)

  def split(f0):
    cpt0 = int(round(f0 * t_chunks / NS / NRING)) * NRING
    cpt1 = -(-(t_chunks - NS * cpt0) // (NS * NRING)) * NRING
    e_pad = NS * (cpt0 + cpt1) * CHUNK
    s = jnp.concatenate(
        [edge_index[0], jnp.zeros((e_pad - e,), jnp.int32)]).reshape(-1, CHUNK)
    d = jnp.concatenate(
        [edge_index[1], jnp.full((e_pad - e,), N_NODES, jnp.int32)]
    ).reshape(-1, CHUNK)
    return cpt0, cpt1, s, d

  xp = jnp.zeros((N_PAD, 32), jnp.float32)
  xp = xp.at[:n, :in_feat].set(x)
  xp = xp.at[:n, 29].set(1.0)

  cpt0a, cpt1a, src_a, dst_a = split(0.60)
  cpt0b, cpt1b, src_b, dst_b = split(0.71)

  z32 = jnp.zeros((N_PAD, 32), jnp.float32)
  z128 = jnp.zeros((N_PAD, 128), jnp.float32)

  def pad_wl1(w):
    return jnp.zeros((MAX_DEG + 1, 32, 128), jnp.float32).at[:, :in_feat].set(w)

  p1, p2, p3, p4 = (params["conv1"], params["conv2"], params["conv3"],
                    params["conv4"])
  wl1, wr1 = pad_wl1(p1["Wl"]), pad_wl1(p1["Wr"])
  w1, b1 = params["lin1"]
  w2, b2 = params["lin2"]
  w2p = jnp.zeros((128, 128), jnp.float32).at[:, :2].set(w2)
  b2p = jnp.zeros((1, 128), jnp.float32).at[0, :2].set(b2)
  b1r = b1.reshape(1, 128)

  seg32 = _make_seg_sum(32, cpt0a, cpt1a)
  seg128 = _make_seg_sum(128, cpt0b, cpt1b)

  h0, h1 = seg32(xp, src_a, dst_a, z32)
  out1, degb = _tc_layer1(h0, h1, xp, wl1, p1["bl"], wr1)

  h0, h1 = seg128(out1, src_b, dst_b, z128)
  out2 = _tc_layer_mid(h0, h1, out1, degb, p2["Wl"], p2["bl"], p2["Wr"])

  h0, h1 = seg128(out2, src_b, dst_b, z128)
  out3 = _tc_layer_mid(h0, h1, out2, degb, p3["Wl"], p3["bl"], p3["Wr"])

  h0, h1 = seg128(out3, src_b, dst_b, z128)
  y = _tc_layer4(h0, h1, out3, degb, out2, p4["Wl"], p4["bl"], p4["Wr"],
                 w1, b1r, w2p, b2p)

  return y[:n, :2]


# speedup vs baseline: 1.1385x; 1.1385x over previous
import functools

import jax
import jax.numpy as jnp
from jax import lax
from jax.experimental import pallas as pl
from jax.experimental.pallas import tpu as pltpu
from jax.experimental.pallas import tpu_sc as plsc

N_NODES = 10000
N_PAD = 10240
NC, NS = 2, 16
NW = NC * NS
CHUNK = 128
RB = 512
MAX_DEG = 5


NRING = 2


def _make_seg_sum(D, cpt0, cpt1):
  groups0, groups1 = cpt0 // NRING, cpt1 // NRING
  rows_per_tile = N_PAD // NS
  mesh = plsc.VectorSubcoreMesh(
      core_axis_name="c", subcore_axis_name="s", num_cores=NC, num_subcores=NS)

  @functools.partial(
      pl.kernel,
      out_type=(jax.ShapeDtypeStruct((N_PAD, D), jnp.float32),
                jax.ShapeDtypeStruct((N_PAD, D), jnp.float32)),
      mesh=mesh,
      compiler_params=pltpu.CompilerParams(use_tc_tiling_on_sc=False),
      scratch_types=[
          pltpu.VMEM_SHARED((N_PAD, D), jnp.float32),
          pltpu.VMEM((2, NRING, CHUNK), jnp.int32),
          pltpu.VMEM((2, NRING, CHUNK), jnp.int32),
          pltpu.VMEM((NRING, CHUNK, D), jnp.float32),
          pltpu.SemaphoreType.DMA,
          pltpu.SemaphoreType.DMA,
          pltpu.SemaphoreType.DMA,
      ],
  )
  def seg(x_hbm, src_hbm, dst_hbm, zero_hbm, h0_hbm, h1_hbm,
          hsh, sidx, didx, ring, gsem, ssem, isem):
    c = lax.axis_index("c")
    s = lax.axis_index("s")
    r0 = s * rows_per_tile
    g0row = jnp.where(c == 0, s * cpt0, NS * cpt0 + s * cpt1)
    groups = jnp.where(c == 0, groups0, groups1)

    pltpu.sync_copy(zero_hbm.at[pl.ds(r0, rows_per_tile)],
                    hsh.at[pl.ds(r0, rows_per_tile)])
    plsc.subcore_barrier()

    def idx_fetch(g, buf):
      row = g0row + g * NRING
      pltpu.async_copy(src_hbm.at[pl.ds(row, NRING)], sidx.at[buf], isem)
      pltpu.async_copy(dst_hbm.at[pl.ds(row, NRING)], didx.at[buf], isem)

    def idx_wait(buf):
      pltpu.make_async_copy(src_hbm.at[pl.ds(g0row, NRING)],
                            sidx.at[buf], isem).wait()
      pltpu.make_async_copy(dst_hbm.at[pl.ds(g0row, NRING)],
                            didx.at[buf], isem).wait()

    def gather(buf, j):
      pltpu.async_copy(x_hbm.at[sidx.at[buf, j]], ring.at[j], gsem)

    def gather_wait(j):
      pltpu.make_async_copy(x_hbm.at[sidx.at[0, 0]], ring.at[j], gsem).wait()

    def scat(buf, j):
      pltpu.async_copy(ring.at[j], hsh.at[didx.at[buf, j]], ssem, add=True)

    def scat_wait(j):
      pltpu.make_async_copy(ring.at[j], hsh.at[didx.at[0, 0]], ssem).wait()

    idx_fetch(0, 0)
    idx_wait(0)
    for j in range(NRING):
      gather(0, j)
    idx_fetch(1, 1)

    def body(g, carry):
      cb = lax.rem(g, 2)
      nb = 1 - cb
      for j in range(NRING):
        gather_wait(j)
        scat(cb, j)

      @pl.when(g + 1 < groups)
      def _():
        idx_wait(nb)
        for j in range(NRING):
          scat_wait(j)
          gather(nb, j)

        @pl.when(g + 2 < groups)
        def _():
          idx_fetch(g + 2, cb)

      return carry

    lax.fori_loop(0, groups, body, 0)
    for j in range(NRING):
      scat_wait(j)
    plsc.subcore_barrier()

    @pl.when(c == 0)
    def _():
      pltpu.sync_copy(hsh.at[pl.ds(r0, rows_per_tile)],
                      h0_hbm.at[pl.ds(r0, rows_per_tile)])

    @pl.when(c == 1)
    def _():
      pltpu.sync_copy(hsh.at[pl.ds(r0, rows_per_tile)],
                      h1_hbm.at[pl.ds(r0, rows_per_tile)])

  return seg


def _bucketed(h, x, degb, wl_ref, bl_ref, wr_ref):
  acc = jnp.zeros((h.shape[0], wl_ref.shape[2]), jnp.float32)
  for d in range(MAX_DEG + 1):
    r = (jnp.dot(h, wl_ref[d], preferred_element_type=jnp.float32)
         + jnp.dot(x, wr_ref[d], preferred_element_type=jnp.float32)
         + bl_ref[d][None, :])
    acc = jnp.where(degb == float(d), r, acc)
  return acc


def _layer1_body(h0, h1, xp, wl, bl, wr, out, degb_out):
  h = h0[...] + h1[...]
  degf = jnp.minimum(h[:, 29:30], float(MAX_DEG))
  degb = jnp.broadcast_to(degf, (h.shape[0], 128))
  out[...] = _bucketed(h, xp[...], degb, wl, bl, wr)
  degb_out[...] = degb


def _layer_mid_body(h0, h1, x, degb, wl, bl, wr, out):
  h = h0[...] + h1[...]
  out[...] = _bucketed(h, x[...], degb[...], wl, bl, wr)


def _layer4_body(h0, h1, x, degb, res2, wl, bl, wr, w1, b1, w2p, b2p, out):
  h = h0[...] + h1[...]
  res = _bucketed(h, x[...], degb[...], wl, bl, wr) + x[...] + res2[...]
  t = jnp.maximum(
      jnp.dot(res, w1[...], preferred_element_type=jnp.float32) + b1[...], 0.0)
  out[...] = (jnp.dot(t, w2p[...], preferred_element_type=jnp.float32)
              + b2p[...])


def _row_spec(dcols):
  return pl.BlockSpec((RB, dcols), lambda g: (g, 0))


def _full_spec(shape):
  nd = len(shape)
  return pl.BlockSpec(shape, lambda g: (0,) * nd)


def _tc_layer1(h0, h1, xp, wl, bl, wr):
  grid = (N_PAD // RB,)
  return pl.pallas_call(
      _layer1_body,
      grid=grid,
      in_specs=[_row_spec(32), _row_spec(32), _row_spec(32),
                _full_spec(wl.shape), _full_spec(bl.shape),
                _full_spec(wr.shape)],
      out_specs=[_row_spec(128), _row_spec(128)],
      out_shape=[jax.ShapeDtypeStruct((N_PAD, 128), jnp.float32)] * 2,
  )(h0, h1, xp, wl, bl, wr)


def _tc_layer_mid(h0, h1, x, degb, wl, bl, wr):
  grid = (N_PAD // RB,)
  return pl.pallas_call(
      _layer_mid_body,
      grid=grid,
      in_specs=[_row_spec(128)] * 4 + [_full_spec(wl.shape),
                                       _full_spec(bl.shape),
                                       _full_spec(wr.shape)],
      out_specs=_row_spec(128),
      out_shape=jax.ShapeDtypeStruct((N_PAD, 128), jnp.float32),
  )(h0, h1, x, degb, wl, bl, wr)


def _tc_layer4(h0, h1, x, degb, res2, wl, bl, wr, w1, b1, w2p, b2p):
  grid = (N_PAD // RB,)
  return pl.pallas_call(
      _layer4_body,
      grid=grid,
      in_specs=[_row_spec(128)] * 5 + [
          _full_spec(wl.shape), _full_spec(bl.shape), _full_spec(wr.shape),
          _full_spec(w1.shape), _full_spec(b1.shape), _full_spec(w2p.shape),
          _full_spec(b2p.shape)],
      out_specs=_row_spec(128),
      out_shape=jax.ShapeDtypeStruct((N_PAD, 128), jnp.float32),
  )(h0, h1, x, degb, res2, wl, bl, wr, w1, b1, w2p, b2p)


def kernel(x, edge_index, params):
  n, in_feat = x.shape
  e = edge_index.shape[1]
  t_chunks = -(-e // CHUNK)

  def split(f0):
    cpt0 = int(round(f0 * t_chunks / NS / NRING)) * NRING
    cpt1 = -(-(t_chunks - NS * cpt0) // (NS * NRING)) * NRING
    return cpt0, cpt1

  xp = jnp.zeros((N_PAD, 32), jnp.float32)
  xp = xp.at[:n, :in_feat].set(x)
  xp = xp.at[:n, 29].set(1.0)

  cpt0a, cpt1a = split(0.60)
  cpt0b, cpt1b = split(0.71)
  rows = NS * max(cpt0a + cpt1a, cpt0b + cpt1b)
  e_pad = rows * CHUNK
  src = jnp.concatenate(
      [edge_index[0], jnp.zeros((e_pad - e,), jnp.int32)]).reshape(-1, CHUNK)
  dst = jnp.concatenate(
      [edge_index[1], jnp.full((e_pad - e,), N_NODES, jnp.int32)]
  ).reshape(-1, CHUNK)

  z32 = jnp.zeros((N_PAD, 32), jnp.float32)
  z128 = jnp.zeros((N_PAD, 128), jnp.float32)

  def pad_wl1(w):
    return jnp.zeros((MAX_DEG + 1, 32, 128), jnp.float32).at[:, :in_feat].set(w)

  p1, p2, p3, p4 = (params["conv1"], params["conv2"], params["conv3"],
                    params["conv4"])
  wl1, wr1 = pad_wl1(p1["Wl"]), pad_wl1(p1["Wr"])
  w1, b1 = params["lin1"]
  w2, b2 = params["lin2"]
  w2p = jnp.zeros((128, 128), jnp.float32).at[:, :2].set(w2)
  b2p = jnp.zeros((1, 128), jnp.float32).at[0, :2].set(b2)
  b1r = b1.reshape(1, 128)

  seg32 = _make_seg_sum(32, cpt0a, cpt1a)
  seg128 = _make_seg_sum(128, cpt0b, cpt1b)

  h0, h1 = seg32(xp, src, dst, z32)
  out1, degb = _tc_layer1(h0, h1, xp, wl1, p1["bl"], wr1)

  h0, h1 = seg128(out1, src, dst, z128)
  out2 = _tc_layer_mid(h0, h1, out1, degb, p2["Wl"], p2["bl"], p2["Wr"])

  h0, h1 = seg128(out2, src, dst, z128)
  out3 = _tc_layer_mid(h0, h1, out2, degb, p3["Wl"], p3["bl"], p3["Wr"])

  h0, h1 = seg128(out3, src, dst, z128)
  y = _tc_layer4(h0, h1, out3, degb, out2, p4["Wl"], p4["bl"], p4["Wr"],
                 w1, b1r, w2p, b2p)

  return y[:n, :2]
